# baseline (speedup 1.0000x reference)
import functools
import jax
import jax.numpy as jnp
from jax import lax
from jax.experimental import pallas as pl
from jax.experimental.pallas import tpu as pltpu
from jax.experimental.pallas import tpu_sc as plsc

_N = 4194304
_NW = 32
_PER_W = _N // _NW
_L = 16
_G = 512
_CHUNK = _G * _L
_NCHUNK = _PER_W // _CHUNK
_NGRP = _N // _L


def _sc_body(x_hbm, params_hbm, out_hbm, xbuf, obuf, params_v):
    wid = lax.axis_index("s") * 2 + lax.axis_index("c")
    base = wid * _PER_W

    pltpu.sync_copy(params_hbm, params_v)
    pv = params_v[...]
    c0 = pv[0]
    c1 = pv[1]
    cs = pv[2]

    one = jnp.full((_L,), 1, jnp.int32)
    zero = jnp.zeros((_L,), jnp.int32)

    def run(pred_fn):
        def chunk_body(g, _):
            row0 = base + g * _CHUNK
            g0 = row0 // _L
            pltpu.sync_copy(x_hbm.at[pl.ds(g0, _G), :, 0], xbuf)

            def inner(j, _):
                xf = xbuf[j, :]
                obuf[pl.ds(j * _L, _L)] = jnp.where(pred_fn(xf), one, zero)
                return 0

            lax.fori_loop(0, _G, inner, 0)
            pltpu.sync_copy(obuf, out_hbm.at[pl.ds(row0, _CHUNK)])
            return 0

        lax.fori_loop(0, _NCHUNK, chunk_body, 0)

    @pl.when(cs == 0.0)
    def _():
        run(lambda xf: xf <= c0)

    @pl.when(cs == 1.0)
    def _():
        run(lambda xf: xf >= c0)

    @pl.when(cs == 2.0)
    def _():
        run(lambda xf: jnp.logical_and(xf >= c0, xf <= c1))

    @pl.when(jnp.logical_and(cs != 0.0,
                             jnp.logical_and(cs != 1.0, cs != 2.0)))
    def _():
        run(lambda xf: jnp.logical_or(xf <= c0, xf >= c1))


def kernel(inputs, cut, case):
    mesh = plsc.VectorSubcoreMesh(core_axis_name="c", subcore_axis_name="s")
    k = functools.partial(
        pl.kernel,
        mesh=mesh,
        out_type=jax.ShapeDtypeStruct((_N,), jnp.int32),
        scratch_types=[
            pltpu.VMEM((_G, _L), jnp.float32),
            pltpu.VMEM((_CHUNK,), jnp.int32),
            pltpu.VMEM((_L,), jnp.float32),
        ],
        compiler_params=pltpu.CompilerParams(use_tc_tiling_on_sc=False),
    )(_sc_body)
    params = (jnp.zeros((_L,), jnp.float32)
              .at[0:2].set(cut)
              .at[2].set(jnp.asarray(case, jnp.float32)))
    return k(inputs.reshape(_NGRP, _L, 4), params)


# speedup vs baseline: 83.2439x; 83.2439x over previous
import functools
import jax
import jax.numpy as jnp
from jax import lax
from jax.experimental import pallas as pl
from jax.experimental.pallas import tpu as pltpu
from jax.experimental.pallas import tpu_sc as plsc

_N = 4194304
_NW = 32
_NT = _N // 128
_TPW = _NT // _NW
_L = 16
_G = 128
_CHUNK = _G * 128
_NCHUNK = _TPW // _G


def _sc_body(x_hbm, params_hbm, out_hbm, xbuf, obuf, params_v):
    wid = lax.axis_index("s") * 2 + lax.axis_index("c")
    tbase = wid * _TPW

    pltpu.sync_copy(params_hbm, params_v)
    pv = params_v[...]
    c0 = pv[0]
    c1 = pv[1]
    cs = pv[2]

    one = jnp.full((_L,), 1, jnp.int32)
    zero = jnp.zeros((_L,), jnp.int32)

    def run(pred_fn):
        def chunk_body(g, _):
            t0 = tbase + g * _G
            pltpu.sync_copy(x_hbm.at[pl.ds(t0, _G), 0, :], xbuf)

            def inner(j, _):
                row = j // 8
                col = (j % 8) * _L
                xf = xbuf[row, pl.ds(col, _L)]
                obuf[pl.ds(j * _L, _L)] = jnp.where(pred_fn(xf), one, zero)
                return 0

            lax.fori_loop(0, _CHUNK // _L, inner, 0)
            pltpu.sync_copy(obuf, out_hbm.at[pl.ds(t0 * 128, _CHUNK)])
            return 0

        lax.fori_loop(0, _NCHUNK, chunk_body, 0)

    @pl.when(cs == 0.0)
    def _():
        run(lambda xf: xf <= c0)

    @pl.when(cs == 1.0)
    def _():
        run(lambda xf: xf >= c0)

    @pl.when(cs == 2.0)
    def _():
        run(lambda xf: jnp.logical_and(xf >= c0, xf <= c1))

    @pl.when(jnp.logical_and(cs != 0.0,
                             jnp.logical_and(cs != 1.0, cs != 2.0)))
    def _():
        run(lambda xf: jnp.logical_or(xf <= c0, xf >= c1))


def kernel(inputs, cut, case):
    mesh = plsc.VectorSubcoreMesh(core_axis_name="c", subcore_axis_name="s")
    k = functools.partial(
        pl.kernel,
        mesh=mesh,
        out_type=jax.ShapeDtypeStruct((_N,), jnp.int32),
        scratch_types=[
            pltpu.VMEM((_G, 128), jnp.float32),
            pltpu.VMEM((_CHUNK,), jnp.int32),
            pltpu.VMEM((_L,), jnp.float32),
        ],
        compiler_params=pltpu.CompilerParams(use_tc_tiling_on_sc=False),
    )(_sc_body)
    params = (jnp.zeros((_L,), jnp.float32)
              .at[0:2].set(cut)
              .at[2].set(jnp.asarray(case, jnp.float32)))
    xview = inputs.T.reshape(4, _NT, 128).transpose(1, 0, 2)
    return k(xview, params)


# speedup vs baseline: 159.3777x; 1.9146x over previous
import functools
import jax
import jax.numpy as jnp
from jax import lax
from jax.experimental import pallas as pl
from jax.experimental.pallas import tpu as pltpu
from jax.experimental.pallas import tpu_sc as plsc

_N = 4194304
_NW = 32
_NT = _N // 128
_TPW = _NT // _NW
_L = 16
_G = 128
_CHUNK = _G * 128
_NCHUNK = _TPW // _G


def _sc_body(x_hbm, params_hbm, out_hbm, xbuf, obuf, params_v):
    wid = lax.axis_index("s") * 2 + lax.axis_index("c")
    tbase = wid * _TPW

    pltpu.sync_copy(params_hbm, params_v)
    pv = params_v[...]
    c0 = pv[0]
    c1 = pv[1]
    cs = pv[2]

    one = jnp.full((_L,), 1, jnp.int32)
    zero = jnp.zeros((_L,), jnp.int32)

    def run(pred_fn):
        def chunk_body(g, _):
            t0 = tbase + g * _G
            pltpu.sync_copy(x_hbm.at[pl.ds(t0, _G), 0, :], xbuf)

            def inner(r, _):
                for q in range(8):
                    xf = xbuf[r, pl.ds(q * _L, _L)]
                    obuf[pl.ds(r * 128 + q * _L, _L)] = (
                        jnp.where(pred_fn(xf), one, zero))
                return 0

            lax.fori_loop(0, _G, inner, 0)
            pltpu.sync_copy(obuf, out_hbm.at[pl.ds(t0 * 128, _CHUNK)])
            return 0

        lax.fori_loop(0, _NCHUNK, chunk_body, 0)

    @pl.when(cs == 0.0)
    def _():
        run(lambda xf: xf <= c0)

    @pl.when(cs == 1.0)
    def _():
        run(lambda xf: xf >= c0)

    @pl.when(cs == 2.0)
    def _():
        run(lambda xf: jnp.logical_and(xf >= c0, xf <= c1))

    @pl.when(jnp.logical_and(cs != 0.0,
                             jnp.logical_and(cs != 1.0, cs != 2.0)))
    def _():
        run(lambda xf: jnp.logical_or(xf <= c0, xf >= c1))


def kernel(inputs, cut, case):
    mesh = plsc.VectorSubcoreMesh(core_axis_name="c", subcore_axis_name="s")
    k = functools.partial(
        pl.kernel,
        mesh=mesh,
        out_type=jax.ShapeDtypeStruct((_N,), jnp.int32),
        scratch_types=[
            pltpu.VMEM((_G, 128), jnp.float32),
            pltpu.VMEM((_CHUNK,), jnp.int32),
            pltpu.VMEM((_L,), jnp.float32),
        ],
        compiler_params=pltpu.CompilerParams(use_tc_tiling_on_sc=False),
    )(_sc_body)
    params = (jnp.zeros((_L,), jnp.float32)
              .at[0:2].set(cut)
              .at[2].set(jnp.asarray(case, jnp.float32)))
    xview = inputs.T.reshape(4, _NT, 128).transpose(1, 0, 2)
    return k(xview, params)


# speedup vs baseline: 188.7687x; 1.1844x over previous
import functools
import jax
import jax.numpy as jnp
from jax import lax
from jax.experimental import pallas as pl
from jax.experimental.pallas import tpu as pltpu
from jax.experimental.pallas import tpu_sc as plsc

_N = 4194304
_NW = 32
_NT = _N // 128
_TPW = _NT // _NW
_L = 16
_G = 128
_CHUNK = _G * 128
_NCHUNK = _TPW // _G


def _sc_body(x_hbm, params_hbm, out_hbm,
             xbuf0, xbuf1, obuf0, obuf1, params_v,
             isem0, isem1, osem0, osem1):
    wid = lax.axis_index("s") * 2 + lax.axis_index("c")
    tbase = wid * _TPW

    pltpu.sync_copy(params_hbm, params_v)
    pv = params_v[...]
    c0 = pv[0]
    c1 = pv[1]
    cs = pv[2]

    one = jnp.full((_L,), 1, jnp.int32)
    zero = jnp.zeros((_L,), jnp.int32)

    xb = (xbuf0, xbuf1)
    ob = (obuf0, obuf1)
    isem = (isem0, isem1)
    osem = (osem0, osem1)

    def in_copy(g, b):
        t0 = tbase + g * _G
        return pltpu.make_async_copy(
            x_hbm.at[pl.ds(t0, _G), 0, :], xb[b], isem[b])

    def out_copy(g, b):
        t0 = tbase + g * _G
        return pltpu.make_async_copy(
            ob[b], out_hbm.at[pl.ds(t0 * 128, _CHUNK)], osem[b])

    def run(pred_fn):
        in_copy(0, 0).start()
        in_copy(1, 1).start()
        for g in range(_NCHUNK):
            b = g % 2
            in_copy(g, b).wait()
            if g >= 2:
                out_copy(g - 2, b).wait()

            def inner(r, _, b=b):
                for q in range(8):
                    xf = xb[b][r, pl.ds(q * _L, _L)]
                    ob[b][pl.ds(r * 128 + q * _L, _L)] = (
                        jnp.where(pred_fn(xf), one, zero))
                return 0

            lax.fori_loop(0, _G, inner, 0)
            out_copy(g, b).start()
            if g + 2 < _NCHUNK:
                in_copy(g + 2, b).start()
        out_copy(_NCHUNK - 2, 0).wait()
        out_copy(_NCHUNK - 1, 1).wait()

    @pl.when(cs == 0.0)
    def _():
        run(lambda xf: xf <= c0)

    @pl.when(cs == 1.0)
    def _():
        run(lambda xf: xf >= c0)

    @pl.when(cs == 2.0)
    def _():
        run(lambda xf: jnp.logical_and(xf >= c0, xf <= c1))

    @pl.when(jnp.logical_and(cs != 0.0,
                             jnp.logical_and(cs != 1.0, cs != 2.0)))
    def _():
        run(lambda xf: jnp.logical_or(xf <= c0, xf >= c1))


def kernel(inputs, cut, case):
    mesh = plsc.VectorSubcoreMesh(core_axis_name="c", subcore_axis_name="s")
    k = functools.partial(
        pl.kernel,
        mesh=mesh,
        out_type=jax.ShapeDtypeStruct((_N,), jnp.int32),
        scratch_types=[
            pltpu.VMEM((_G, 128), jnp.float32),
            pltpu.VMEM((_G, 128), jnp.float32),
            pltpu.VMEM((_CHUNK,), jnp.int32),
            pltpu.VMEM((_CHUNK,), jnp.int32),
            pltpu.VMEM((_L,), jnp.float32),
            pltpu.SemaphoreType.DMA,
            pltpu.SemaphoreType.DMA,
            pltpu.SemaphoreType.DMA,
            pltpu.SemaphoreType.DMA,
        ],
        compiler_params=pltpu.CompilerParams(use_tc_tiling_on_sc=False),
    )(_sc_body)
    params = (jnp.zeros((_L,), jnp.float32)
              .at[0:2].set(cut)
              .at[2].set(jnp.asarray(case, jnp.float32)))
    xview = inputs.T.reshape(4, _NT, 128).transpose(1, 0, 2)
    return k(xview, params)


# speedup vs baseline: 192.1580x; 1.0180x over previous
import functools
import jax
import jax.numpy as jnp
from jax import lax
from jax.experimental import pallas as pl
from jax.experimental.pallas import tpu as pltpu
from jax.experimental.pallas import tpu_sc as plsc

_N = 4194304
_NW = 32
_NT = _N // 128
_TPW = _NT // _NW
_L = 16
_G = 128
_CHUNK = _G * 128
_NCHUNK = _TPW // _G


def _sc_body(x_hbm, cut_hbm, case_hbm, out_hbm,
             xbuf0, xbuf1, obuf0, obuf1, params_v,
             isem0, isem1, osem0, osem1):
    wid = lax.axis_index("s") * 2 + lax.axis_index("c")
    tbase = wid * _TPW

    one = jnp.full((_L,), 1, jnp.int32)
    zero = jnp.zeros((_L,), jnp.int32)

    xb = (xbuf0, xbuf1)
    ob = (obuf0, obuf1)
    isem = (isem0, isem1)
    osem = (osem0, osem1)

    def in_copy(g, b):
        t0 = tbase + g * _G
        return pltpu.make_async_copy(
            x_hbm.at[pl.ds(t0, _G), 0, :], xb[b], isem[b])

    def out_copy(g, b):
        t0 = tbase + g * _G
        return pltpu.make_async_copy(
            ob[b], out_hbm.at[pl.ds(t0 * 128, _CHUNK)], osem[b])

    in_copy(0, 0).start()
    in_copy(1, 1).start()

    pltpu.sync_copy(cut_hbm, params_v.at[pl.ds(0, 2)])
    pltpu.sync_copy(case_hbm, params_v.at[pl.ds(8, 1)])
    pv = params_v[...]
    c0 = pv[0]
    c1 = pv[1]
    cs = pv[8]

    def run(pred_fn):
        for g in range(_NCHUNK):
            b = g % 2
            in_copy(g, b).wait()
            if g >= 2:
                out_copy(g - 2, b).wait()

            def inner(r, _, b=b):
                for q in range(8):
                    xf = xb[b][r, pl.ds(q * _L, _L)]
                    ob[b][pl.ds(r * 128 + q * _L, _L)] = (
                        jnp.where(pred_fn(xf), one, zero))
                return 0

            lax.fori_loop(0, _G, inner, 0)
            out_copy(g, b).start()
            if g + 2 < _NCHUNK:
                in_copy(g + 2, b).start()
        out_copy(_NCHUNK - 2, 0).wait()
        out_copy(_NCHUNK - 1, 1).wait()

    @pl.when(cs == 0.0)
    def _():
        run(lambda xf: xf <= c0)

    @pl.when(cs == 1.0)
    def _():
        run(lambda xf: xf >= c0)

    @pl.when(cs == 2.0)
    def _():
        run(lambda xf: jnp.logical_and(xf >= c0, xf <= c1))

    @pl.when(jnp.logical_and(cs != 0.0,
                             jnp.logical_and(cs != 1.0, cs != 2.0)))
    def _():
        run(lambda xf: jnp.logical_or(xf <= c0, xf >= c1))


def kernel(inputs, cut, case):
    mesh = plsc.VectorSubcoreMesh(core_axis_name="c", subcore_axis_name="s")
    k = functools.partial(
        pl.kernel,
        mesh=mesh,
        out_type=jax.ShapeDtypeStruct((_N,), jnp.int32),
        scratch_types=[
            pltpu.VMEM((_G, 128), jnp.float32),
            pltpu.VMEM((_G, 128), jnp.float32),
            pltpu.VMEM((_CHUNK,), jnp.int32),
            pltpu.VMEM((_CHUNK,), jnp.int32),
            pltpu.VMEM((_L,), jnp.float32),
            pltpu.SemaphoreType.DMA,
            pltpu.SemaphoreType.DMA,
            pltpu.SemaphoreType.DMA,
            pltpu.SemaphoreType.DMA,
        ],
        compiler_params=pltpu.CompilerParams(use_tc_tiling_on_sc=False),
    )(_sc_body)
    xview = inputs.T.reshape(4, _NT, 128).transpose(1, 0, 2)
    return k(xview, cut, jnp.asarray(case, jnp.float32).reshape(1))
